# baseline (speedup 1.0000x reference)
import functools

import jax
import jax.numpy as jnp
from jax import lax
from jax.experimental import pallas as pl
from jax.experimental.pallas import tpu as pltpu
from jax.experimental.pallas import tpu_sc as plsc

N = 50000
D = 128
NB = 2

NBLK = 4
RB = 12544
NPAD = NBLK * RB
TPB = RB // 16

NE = 7 * 80000
NTILE = 16
EPT = 35840
NE_PAD = NTILE * EPT
CH = 128
NCH = EPT // CH

ROW_BLK = 400


def _spmm_body(rows_hbm, cols_hbm, vals_hbm, emb_hbm, zeros_hbm, acc_hbm,
               acc_sp, rows_v, cols_v, vals_v, lrows_v, mvals_v, gath_v, sem):
    c = lax.axis_index("c")
    s = lax.axis_index("s")
    for p in range(NBLK // 2):
        blk = c * (NBLK // 2) + p
        lo = blk * RB
        pltpu.sync_copy(zeros_hbm, acc_sp.at[pl.ds(s * TPB, TPB)])
        plsc.subcore_barrier()

        base0 = s * EPT

        def chunk_body(k, _):
            base = base0 + k * CH
            pltpu.sync_copy(rows_hbm.at[pl.ds(base, CH)], rows_v)
            pltpu.sync_copy(cols_hbm.at[pl.ds(base, CH)], cols_v)
            pltpu.sync_copy(vals_hbm.at[pl.ds(base, CH)], vals_v)
            for j in range(CH // 16):
                sl = pl.ds(j * 16, 16)
                r = rows_v[sl]
                inb = (r >= lo) & (r < lo + RB)
                lrows_v[sl] = jnp.where(inb, r - lo, 0)
                mvals_v[sl] = jnp.where(inb, vals_v[sl], 0.0)
            pltpu.async_copy(emb_hbm.at[cols_v], gath_v, sem).wait()

            def scale_body(g, _):
                mv16 = mvals_v[pl.ds(g * 16, 16)]
                for i in range(16):
                    e = g * 16 + i
                    mv = mv16[i]
                    for jj in range(D // 16):
                        sl2 = pl.ds(jj * 16, 16)
                        gath_v[e, sl2] = gath_v[e, sl2] * mv
                return 0

            lax.fori_loop(0, CH // 16, scale_body, 0)
            pltpu.sync_copy(gath_v, acc_sp.at[lrows_v], add=True)
            return 0

        lax.fori_loop(0, NCH, chunk_body, 0)
        plsc.subcore_barrier()
        pltpu.sync_copy(acc_sp.at[pl.ds(s * TPB, TPB)],
                        acc_hbm.at[pl.ds(lo + s * TPB, TPB)])


_spmm = functools.partial(
    pl.kernel,
    out_type=jax.ShapeDtypeStruct((NPAD, D), jnp.float32),
    mesh=plsc.VectorSubcoreMesh(core_axis_name="c", subcore_axis_name="s"),
    scratch_types=[
        pltpu.VMEM_SHARED((RB, D), jnp.float32),
        pltpu.VMEM((CH,), jnp.int32),
        pltpu.VMEM((CH,), jnp.int32),
        pltpu.VMEM((CH,), jnp.float32),
        pltpu.VMEM((CH,), jnp.int32),
        pltpu.VMEM((CH,), jnp.float32),
        pltpu.VMEM((CH, D), jnp.float32),
        pltpu.SemaphoreType.DMA,
    ],
)(_spmm_body)


def _transform_body(cvec_ref, w_ref, acc_ref, out_ref, *, last):
    w = cvec_ref[0, 0] * w_ref[0] + cvec_ref[0, 1] * w_ref[1]
    y = jnp.dot(acc_ref[...], w.T, preferred_element_type=jnp.float32)
    y = jnp.maximum(y, 0.0)
    if last:
        nrm = jnp.sqrt(jnp.sum(y * y, axis=1, keepdims=True))
        y = y / jnp.maximum(nrm, 1e-12)
    out_ref[...] = y


def _transform(acc, weights_l, coeff_l, last):
    cvec = coeff_l.reshape(1, NB)
    grid = N // ROW_BLK
    return pl.pallas_call(
        functools.partial(_transform_body, last=last),
        grid=(grid,),
        in_specs=[
            pl.BlockSpec((1, NB), lambda i: (0, 0)),
            pl.BlockSpec((NB, D, D), lambda i: (0, 0, 0)),
            pl.BlockSpec((ROW_BLK, D), lambda i: (i, 0)),
        ],
        out_specs=pl.BlockSpec((ROW_BLK, D), lambda i: (i, 0)),
        out_shape=jax.ShapeDtypeStruct((N, D), jnp.float32),
    )(cvec, weights_l, acc)


def kernel(adj_row, adj_col, adj_val, ent_emb, basis_weights, basis_coeff):
    pad = NE_PAD - NE
    rows = jnp.concatenate([adj_row.reshape(-1).astype(jnp.int32),
                            jnp.zeros((pad,), jnp.int32)])
    cols = jnp.concatenate([adj_col.reshape(-1).astype(jnp.int32),
                            jnp.zeros((pad,), jnp.int32)])
    vals = jnp.concatenate([adj_val.reshape(-1),
                            jnp.zeros((pad,), jnp.float32)])
    zeros = jnp.zeros((TPB, D), jnp.float32)
    emb = ent_emb
    L = basis_weights.shape[0]
    for l in range(L):
        acc = _spmm(rows, cols, vals, emb, zeros)[:N]
        emb = _transform(acc, basis_weights[l], basis_coeff[l, l], last=(l == L - 1))
    return emb


# speedup vs baseline: 3.4626x; 3.4626x over previous
import functools

import jax
import jax.numpy as jnp
from jax import lax
from jax.experimental import pallas as pl
from jax.experimental.pallas import tpu as pltpu
from jax.experimental.pallas import tpu_sc as plsc

N = 50000
D = 128
NB = 2

SH = 4
SW = D // SH
NR = 50048
TPR = NR // 16

NE = 7 * 80000
CH = 128
NCHT = 280
EPT = NCHT * CH
NE_PAD = 16 * EPT
NE_LEN = NE_PAD + 2 * CH

ROW_BLK = 400


def _spmm_body(rows_hbm, cv_hbm, emb_hbm, zeros_hbm, acc_hbm, acc_sp,
               rows0, rows1, cv0, cv1, cidx0, cidx1, srow0, srow1, g0, g1,
               msem0, msem1, gsem0, gsem1, ssem0, ssem1):
    c = lax.axis_index("c")
    s = lax.axis_index("s")
    rows_v = (rows0, rows1)
    cv_v = (cv0, cv1)
    cidx_v = (cidx0, cidx1)
    srow_v = (srow0, srow1)
    gath_v = (g0, g1)
    msem = (msem0, msem1)
    gsem = (gsem0, gsem1)
    ssem = (ssem0, ssem1)
    qbase = s * NCHT

    def meta_start(b, k):
        q = qbase + k
        pltpu.async_copy(rows_hbm.at[pl.ds(q * CH, CH)], rows_v[b], msem[b])
        pltpu.async_copy(cv_hbm.at[pl.ds(q * 2 * CH, 2 * CH)], cv_v[b], msem[b])

    def meta_wait(b, k):
        q = qbase + k
        pltpu.make_async_copy(
            rows_hbm.at[pl.ds(q * CH, CH)], rows_v[b], msem[b]).wait()
        pltpu.make_async_copy(
            cv_hbm.at[pl.ds(q * 2 * CH, 2 * CH)], cv_v[b], msem[b]).wait()

    def gather_start(b, blkbase):
        for g in range(CH // 16):
            sl = pl.ds(g * 16, 16)
            cidx_v[b][sl] = cv_v[b][sl] + blkbase
        pltpu.async_copy(emb_hbm.at[cidx_v[b]], gath_v[b], gsem[b])

    def gather_wait(b):
        pltpu.make_async_copy(emb_hbm.at[cidx_v[b]], gath_v[b], gsem[b]).wait()

    def scale_copy(b):
        for g in range(CH // 16):
            sl = pl.ds(g * 16, 16)
            v16 = lax.bitcast_convert_type(cv_v[b][pl.ds(CH + g * 16, 16)],
                                           jnp.float32)
            srow_v[b][sl] = rows_v[b][sl]
            for i in range(16):
                e = g * 16 + i
                mv = v16[i]
                for jj in range(SW // 16):
                    sl2 = pl.ds(jj * 16, 16)
                    gath_v[b][e, sl2] = gath_v[b][e, sl2] * mv

    def scatter_start(b):
        pltpu.async_copy(gath_v[b], acc_sp.at[srow_v[b]], ssem[b], add=True)

    def scatter_wait(b):
        pltpu.make_async_copy(gath_v[b], acc_sp.at[srow_v[b]], ssem[b]).wait()

    def steady(b, kk, first=False):
        bo = 1 - b
        gather_wait(b)
        meta_wait(bo, kk + 1)
        if not first:
            scatter_wait(bo)
        gather_start(bo, blkbase)
        scale_copy(b)
        meta_start(b, kk + 2)
        scatter_start(b)

    for p in range(2):
        blk = c * 2 + p
        blkbase = blk * NR
        pltpu.sync_copy(zeros_hbm, acc_sp.at[pl.ds(s * TPR, TPR)])
        plsc.subcore_barrier()

        meta_start(0, 0)
        meta_start(1, 1)
        meta_wait(0, 0)
        gather_start(0, blkbase)
        steady(0, 0, first=True)

        def pair_body(i, _):
            kk = 2 * i + 1
            steady(1, kk)
            steady(0, kk + 1)
            return 0

        lax.fori_loop(0, (NCHT - 2) // 2, pair_body, 0)

        steady(1, NCHT - 1)
        gather_wait(0)
        meta_wait(1, NCHT + 1)
        scatter_wait(1)

        plsc.subcore_barrier()
        pltpu.sync_copy(acc_sp.at[pl.ds(s * TPR, TPR)],
                        acc_hbm.at[blk, pl.ds(s * TPR, TPR)])


_spmm = functools.partial(
    pl.kernel,
    out_type=jax.ShapeDtypeStruct((SH, NR, SW), jnp.float32),
    mesh=plsc.VectorSubcoreMesh(core_axis_name="c", subcore_axis_name="s"),
    compiler_params=pltpu.CompilerParams(use_tc_tiling_on_sc=False),
    scratch_types=[
        pltpu.VMEM_SHARED((NR, SW), jnp.float32),
        pltpu.VMEM((CH,), jnp.int32),
        pltpu.VMEM((CH,), jnp.int32),
        pltpu.VMEM((2 * CH,), jnp.int32),
        pltpu.VMEM((2 * CH,), jnp.int32),
        pltpu.VMEM((CH,), jnp.int32),
        pltpu.VMEM((CH,), jnp.int32),
        pltpu.VMEM((CH,), jnp.int32),
        pltpu.VMEM((CH,), jnp.int32),
        pltpu.VMEM((CH, SW), jnp.float32),
        pltpu.VMEM((CH, SW), jnp.float32),
        pltpu.SemaphoreType.DMA,
        pltpu.SemaphoreType.DMA,
        pltpu.SemaphoreType.DMA,
        pltpu.SemaphoreType.DMA,
        pltpu.SemaphoreType.DMA,
        pltpu.SemaphoreType.DMA,
    ],
)(_spmm_body)


def _transform_body(cvec_ref, w_ref, a0, a1, a2, a3, out_ref, *, last):
    w = cvec_ref[0, 0] * w_ref[0] + cvec_ref[0, 1] * w_ref[1]
    acc = jnp.concatenate([a0[0], a1[0], a2[0], a3[0]], axis=1)
    y = jnp.dot(acc, w.T, preferred_element_type=jnp.float32)
    y = jnp.maximum(y, 0.0)
    if last:
        nrm = jnp.sqrt(jnp.sum(y * y, axis=1, keepdims=True))
        y = y / jnp.maximum(nrm, 1e-12)
        out_ref[...] = y
    else:
        for b in range(SH):
            out_ref[b] = y[:, b * SW:(b + 1) * SW]


def _transform(acc_sh, weights_l, coeff_l, last):
    cvec = coeff_l.reshape(1, NB)
    grid = N // ROW_BLK
    in_specs = [
        pl.BlockSpec((1, NB), lambda i: (0, 0)),
        pl.BlockSpec((NB, D, D), lambda i: (0, 0, 0)),
    ] + [
        pl.BlockSpec((1, ROW_BLK, SW), functools.partial(
            lambda i, b: (b, i, 0), b=b))
        for b in range(SH)
    ]
    if last:
        out_spec = pl.BlockSpec((ROW_BLK, D), lambda i: (i, 0))
        out_shape = jax.ShapeDtypeStruct((N, D), jnp.float32)
    else:
        out_spec = pl.BlockSpec((SH, ROW_BLK, SW), lambda i: (0, i, 0))
        out_shape = jax.ShapeDtypeStruct((SH, NR, SW), jnp.float32)
    return pl.pallas_call(
        functools.partial(_transform_body, last=last),
        grid=(grid,),
        in_specs=in_specs,
        out_specs=out_spec,
        out_shape=out_shape,
    )(cvec, weights_l, acc_sh, acc_sh, acc_sh, acc_sh)


def kernel(adj_row, adj_col, adj_val, ent_emb, basis_weights, basis_coeff):
    pad = NE_LEN - NE
    rows = jnp.concatenate([adj_row.reshape(-1).astype(jnp.int32),
                            jnp.zeros((pad,), jnp.int32)])
    cols = jnp.concatenate([adj_col.reshape(-1).astype(jnp.int32),
                            jnp.zeros((pad,), jnp.int32)])
    vals = jnp.concatenate([adj_val.reshape(-1),
                            jnp.zeros((pad,), jnp.float32)])
    cv = jnp.stack([cols.reshape(-1, CH),
                    lax.bitcast_convert_type(vals, jnp.int32).reshape(-1, CH)],
                   axis=1).reshape(-1)
    zeros = jnp.zeros((TPR, SW), jnp.float32)
    emb_sh = jnp.pad(ent_emb.reshape(N, SH, SW).transpose(1, 0, 2),
                     ((0, 0), (0, NR - N), (0, 0)))
    emb_flat = emb_sh.reshape(SH * NR, SW)
    L = basis_weights.shape[0]
    out = None
    for l in range(L):
        acc_sh = _spmm(rows, cv, emb_flat, zeros)
        res = _transform(acc_sh, basis_weights[l], basis_coeff[l, l],
                         last=(l == L - 1))
        if l == L - 1:
            out = res
        else:
            emb_flat = res.reshape(SH * NR, SW)
    return out


# speedup vs baseline: 4.0808x; 1.1785x over previous
import functools

import jax
import jax.numpy as jnp
from jax import lax
from jax.experimental import pallas as pl
from jax.experimental.pallas import tpu as pltpu
from jax.experimental.pallas import tpu_sc as plsc

N = 50000
D = 128
NB = 2

SH = 4
SW = D // SH
NR = 50048
TPR = NR // 16

NE = 7 * 80000
CH = 128
NCHT = 280
EPT = NCHT * CH
NE_PAD = 16 * EPT
NE_LEN = NE_PAD + 4 * CH
NBUF = 4

ROW_BLK = 400


def _spmm_body(rows_hbm, cv_hbm, emb_hbm, zeros_hbm, acc_hbm, acc_sp,
               *bufs):
    c = lax.axis_index("c")
    s = lax.axis_index("s")
    rows_v = bufs[0:4]
    cv_v = bufs[4:8]
    cidx_v = bufs[8:12]
    srow_v = bufs[12:16]
    gath_v = bufs[16:20]
    msem = bufs[20:24]
    gsem = bufs[24:28]
    ssem = bufs[28:32]
    qbase = s * NCHT

    def meta_start(b, k):
        q = qbase + k
        pltpu.async_copy(rows_hbm.at[pl.ds(q * CH, CH)], rows_v[b], msem[b])
        pltpu.async_copy(cv_hbm.at[pl.ds(q * 2 * CH, 2 * CH)], cv_v[b], msem[b])

    def meta_wait(b, k):
        q = qbase + k
        pltpu.make_async_copy(
            rows_hbm.at[pl.ds(q * CH, CH)], rows_v[b], msem[b]).wait()
        pltpu.make_async_copy(
            cv_hbm.at[pl.ds(q * 2 * CH, 2 * CH)], cv_v[b], msem[b]).wait()

    def gather_start(b, blkbase):
        for g in range(CH // 16):
            sl = pl.ds(g * 16, 16)
            cidx_v[b][sl] = cv_v[b][sl] + blkbase
        pltpu.async_copy(emb_hbm.at[cidx_v[b]], gath_v[b], gsem[b])

    def gather_wait(b):
        pltpu.make_async_copy(emb_hbm.at[cidx_v[b]], gath_v[b], gsem[b]).wait()

    def scale_copy(b):
        for g in range(CH // 16):
            sl = pl.ds(g * 16, 16)
            v16 = lax.bitcast_convert_type(cv_v[b][pl.ds(CH + g * 16, 16)],
                                           jnp.float32)
            srow_v[b][sl] = rows_v[b][sl]
            for i in range(16):
                e = g * 16 + i
                mv = v16[i]
                for jj in range(SW // 16):
                    sl2 = pl.ds(jj * 16, 16)
                    gath_v[b][e, sl2] = gath_v[b][e, sl2] * mv

    def scatter_start(b):
        pltpu.async_copy(gath_v[b], acc_sp.at[srow_v[b]], ssem[b], add=True)

    def scatter_wait(b):
        pltpu.make_async_copy(gath_v[b], acc_sp.at[srow_v[b]], ssem[b]).wait()

    def chunk(kk, b, blkbase, first=False):
        b2 = (b + 2) % NBUF
        gather_wait(b)
        scale_copy(b)
        scatter_start(b)
        if not first:
            scatter_wait(b2)
        meta_wait(b2, kk + 2)
        gather_start(b2, blkbase)
        meta_start(b, kk + 4)

    for p in range(2):
        blk = c * 2 + p
        blkbase = blk * NR
        pltpu.sync_copy(zeros_hbm, acc_sp.at[pl.ds(s * TPR, TPR)])
        plsc.subcore_barrier()

        for b in range(NBUF):
            meta_start(b, b)
        for b in range(2):
            meta_wait(b, b)
            gather_start(b, blkbase)
        chunk(0, 0, blkbase, first=True)
        chunk(1, 1, blkbase, first=True)

        def quad_body(i, _):
            kk = 4 * i + 2
            chunk(kk, 2, blkbase)
            chunk(kk + 1, 3, blkbase)
            chunk(kk + 2, 0, blkbase)
            chunk(kk + 3, 1, blkbase)
            return 0

        lax.fori_loop(0, (NCHT - 4) // 4, quad_body, 0)

        chunk(NCHT - 2, 2, blkbase)
        chunk(NCHT - 1, 3, blkbase)
        gather_wait(0)
        gather_wait(1)
        meta_wait(2, NCHT + 2)
        meta_wait(3, NCHT + 3)
        scatter_wait(2)
        scatter_wait(3)

        plsc.subcore_barrier()
        pltpu.sync_copy(acc_sp.at[pl.ds(s * TPR, TPR)],
                        acc_hbm.at[blk, pl.ds(s * TPR, TPR)])


_spmm = functools.partial(
    pl.kernel,
    out_type=jax.ShapeDtypeStruct((SH, NR, SW), jnp.float32),
    mesh=plsc.VectorSubcoreMesh(core_axis_name="c", subcore_axis_name="s"),
    compiler_params=pltpu.CompilerParams(use_tc_tiling_on_sc=False),
    scratch_types=(
        [pltpu.VMEM_SHARED((NR, SW), jnp.float32)]
        + [pltpu.VMEM((CH,), jnp.int32)] * NBUF
        + [pltpu.VMEM((2 * CH,), jnp.int32)] * NBUF
        + [pltpu.VMEM((CH,), jnp.int32)] * NBUF
        + [pltpu.VMEM((CH,), jnp.int32)] * NBUF
        + [pltpu.VMEM((CH, SW), jnp.float32)] * NBUF
        + [pltpu.SemaphoreType.DMA] * (3 * NBUF)
    ),
)(_spmm_body)


def _transform_body(cvec_ref, w_ref, acc_ref, out_ref, *, last):
    w = cvec_ref[0, 0] * w_ref[0] + cvec_ref[0, 1] * w_ref[1]
    acc = jnp.concatenate([acc_ref[b] for b in range(SH)], axis=1)
    y = jnp.dot(acc, w.T, preferred_element_type=jnp.float32)
    y = jnp.maximum(y, 0.0)
    if last:
        nrm = jnp.sqrt(jnp.sum(y * y, axis=1, keepdims=True))
        y = y / jnp.maximum(nrm, 1e-12)
        out_ref[...] = y
    else:
        for b in range(SH):
            out_ref[b] = y[:, b * SW:(b + 1) * SW]


def _transform(acc_sh, weights_l, coeff_l, last):
    cvec = coeff_l.reshape(1, NB)
    grid = N // ROW_BLK
    in_specs = [
        pl.BlockSpec((1, NB), lambda i: (0, 0)),
        pl.BlockSpec((NB, D, D), lambda i: (0, 0, 0)),
        pl.BlockSpec((SH, ROW_BLK, SW), lambda i: (0, i, 0)),
    ]
    if last:
        out_spec = pl.BlockSpec((ROW_BLK, D), lambda i: (i, 0))
        out_shape = jax.ShapeDtypeStruct((N, D), jnp.float32)
    else:
        out_spec = pl.BlockSpec((SH, ROW_BLK, SW), lambda i: (0, i, 0))
        out_shape = jax.ShapeDtypeStruct((SH, NR, SW), jnp.float32)
    return pl.pallas_call(
        functools.partial(_transform_body, last=last),
        grid=(grid,),
        in_specs=in_specs,
        out_specs=out_spec,
        out_shape=out_shape,
    )(cvec, weights_l, acc_sh)


def kernel(adj_row, adj_col, adj_val, ent_emb, basis_weights, basis_coeff):
    pad = NE_LEN - NE
    rows = jnp.concatenate([adj_row.reshape(-1).astype(jnp.int32),
                            jnp.zeros((pad,), jnp.int32)])
    cols = jnp.concatenate([adj_col.reshape(-1).astype(jnp.int32),
                            jnp.zeros((pad,), jnp.int32)])
    vals = jnp.concatenate([adj_val.reshape(-1),
                            jnp.zeros((pad,), jnp.float32)])
    cv = jnp.stack([cols.reshape(-1, CH),
                    lax.bitcast_convert_type(vals, jnp.int32).reshape(-1, CH)],
                   axis=1).reshape(-1)
    zeros = jnp.zeros((TPR, SW), jnp.float32)
    emb_sh = jnp.pad(ent_emb.reshape(N, SH, SW).transpose(1, 0, 2),
                     ((0, 0), (0, NR - N), (0, 0)))
    emb_flat = emb_sh.reshape(SH * NR, SW)
    L = basis_weights.shape[0]
    out = None
    for l in range(L):
        acc_sh = _spmm(rows, cv, emb_flat, zeros)
        res = _transform(acc_sh, basis_weights[l], basis_coeff[l, l],
                         last=(l == L - 1))
        if l == L - 1:
            out = res
        else:
            emb_flat = res.reshape(SH * NR, SW)
    return out


# speedup vs baseline: 4.2753x; 1.0477x over previous
import functools

import jax
import jax.numpy as jnp
from jax import lax
from jax.experimental import pallas as pl
from jax.experimental.pallas import tpu as pltpu
from jax.experimental.pallas import tpu_sc as plsc

N = 50000
D = 128
NB = 2

SH = 4
SW = D // SH
TPR = N // 16

NE = 7 * 80000
CH = 128
NCHT = 280
EPT = NCHT * CH
NE_PAD = 16 * EPT
NE_LEN = NE_PAD + 4 * CH
NBUF = 4

ROW_BLK = 400


def _spmm_body(rows_hbm, cv_hbm, emb_hbm, zeros_hbm, acc_hbm, acc_sp,
               *bufs):
    c = lax.axis_index("c")
    s = lax.axis_index("s")
    rows_v = bufs[0:4]
    cv_v = bufs[4:8]
    cidx_v = bufs[8:12]
    srow_v = bufs[12:16]
    gath_v = bufs[16:20]
    msem = bufs[20:24]
    gsem = bufs[24:28]
    ssem = bufs[28:32]
    qbase = s * NCHT
    acc2d = acc_hbm
    zeros2d = zeros_hbm

    def meta_start(b, k):
        q = qbase + k
        pltpu.async_copy(rows_hbm.at[pl.ds(q * CH, CH)], rows_v[b], msem[b])
        pltpu.async_copy(cv_hbm.at[pl.ds(q * 2 * CH, 2 * CH)], cv_v[b], msem[b])

    def meta_wait(b, k):
        q = qbase + k
        pltpu.make_async_copy(
            rows_hbm.at[pl.ds(q * CH, CH)], rows_v[b], msem[b]).wait()
        pltpu.make_async_copy(
            cv_hbm.at[pl.ds(q * 2 * CH, 2 * CH)], cv_v[b], msem[b]).wait()

    def gather_start(b, blk):
        for g in range(CH // 16):
            sl = pl.ds(g * 16, 16)
            cidx_v[b][sl] = cv_v[b][sl] * SH + blk
        pltpu.async_copy(emb_hbm.at[cidx_v[b]], gath_v[b], gsem[b])

    def gather_wait(b):
        pltpu.make_async_copy(emb_hbm.at[cidx_v[b]], gath_v[b], gsem[b]).wait()

    def scale_copy(b):
        for g in range(CH // 16):
            sl = pl.ds(g * 16, 16)
            v16 = lax.bitcast_convert_type(cv_v[b][pl.ds(CH + g * 16, 16)],
                                           jnp.float32)
            srow_v[b][sl] = rows_v[b][sl]
            for i in range(16):
                e = g * 16 + i
                mv = v16[i]
                for jj in range(SW // 16):
                    sl2 = pl.ds(jj * 16, 16)
                    gath_v[b][e, sl2] = gath_v[b][e, sl2] * mv

    def scatter_start(b):
        pltpu.async_copy(gath_v[b], acc_sp.at[srow_v[b]], ssem[b], add=True)

    def scatter_wait(b):
        pltpu.make_async_copy(gath_v[b], acc_sp.at[srow_v[b]], ssem[b]).wait()

    def chunk(kk, b, blk, first=False):
        b2 = (b + 2) % NBUF
        gather_wait(b)
        scale_copy(b)
        scatter_start(b)
        if not first:
            scatter_wait(b2)
        meta_wait(b2, kk + 2)
        gather_start(b2, blk)
        meta_start(b, kk + 4)

    for p in range(2):
        blk = c * 2 + p
        pltpu.sync_copy(zeros2d, acc_sp.at[pl.ds(s * TPR, TPR)])
        plsc.subcore_barrier()

        for b in range(NBUF):
            meta_start(b, b)
        for b in range(2):
            meta_wait(b, b)
            gather_start(b, blk)
        chunk(0, 0, blk, first=True)
        chunk(1, 1, blk, first=True)

        def quad_body(i, _):
            kk = 4 * i + 2
            chunk(kk, 2, blk)
            chunk(kk + 1, 3, blk)
            chunk(kk + 2, 0, blk)
            chunk(kk + 3, 1, blk)
            return 0

        lax.fori_loop(0, (NCHT - 4) // 4, quad_body, 0)

        chunk(NCHT - 2, 2, blk)
        chunk(NCHT - 1, 3, blk)
        gather_wait(0)
        gather_wait(1)
        meta_wait(2, NCHT + 2)
        meta_wait(3, NCHT + 3)
        scatter_wait(2)
        scatter_wait(3)

        plsc.subcore_barrier()
        pltpu.sync_copy(acc_sp.at[pl.ds(s * TPR, TPR)],
                        acc2d.at[pl.ds(s * TPR, TPR), pl.ds(blk * SW, SW)])


_spmm = functools.partial(
    pl.kernel,
    out_type=jax.ShapeDtypeStruct((N, D), jnp.float32),
    mesh=plsc.VectorSubcoreMesh(core_axis_name="c", subcore_axis_name="s"),
    compiler_params=pltpu.CompilerParams(use_tc_tiling_on_sc=False),
    scratch_types=(
        [pltpu.VMEM_SHARED((N, SW), jnp.float32)]
        + [pltpu.VMEM((CH,), jnp.int32)] * NBUF
        + [pltpu.VMEM((2 * CH,), jnp.int32)] * NBUF
        + [pltpu.VMEM((CH,), jnp.int32)] * NBUF
        + [pltpu.VMEM((CH,), jnp.int32)] * NBUF
        + [pltpu.VMEM((CH, SW), jnp.float32)] * NBUF
        + [pltpu.SemaphoreType.DMA] * (3 * NBUF)
    ),
)(_spmm_body)


def _transform_body(cvec_ref, w_ref, acc_ref, out_ref, *, last):
    w = cvec_ref[0, 0] * w_ref[0] + cvec_ref[0, 1] * w_ref[1]
    y = jnp.dot(acc_ref[...], w.T, preferred_element_type=jnp.float32)
    y = jnp.maximum(y, 0.0)
    if last:
        nrm = jnp.sqrt(jnp.sum(y * y, axis=1, keepdims=True))
        y = y / jnp.maximum(nrm, 1e-12)
    out_ref[...] = y


def _transform(acc, weights_l, coeff_l, last):
    cvec = coeff_l.reshape(1, NB)
    grid = N // ROW_BLK
    return pl.pallas_call(
        functools.partial(_transform_body, last=last),
        grid=(grid,),
        in_specs=[
            pl.BlockSpec((1, NB), lambda i: (0, 0)),
            pl.BlockSpec((NB, D, D), lambda i: (0, 0, 0)),
            pl.BlockSpec((ROW_BLK, D), lambda i: (i, 0)),
        ],
        out_specs=pl.BlockSpec((ROW_BLK, D), lambda i: (i, 0)),
        out_shape=jax.ShapeDtypeStruct((N, D), jnp.float32),
    )(cvec, weights_l, acc)


def kernel(adj_row, adj_col, adj_val, ent_emb, basis_weights, basis_coeff):
    pad = NE_LEN - NE
    rows = jnp.concatenate([adj_row.reshape(-1).astype(jnp.int32),
                            jnp.zeros((pad,), jnp.int32)])
    cols = jnp.concatenate([adj_col.reshape(-1).astype(jnp.int32),
                            jnp.zeros((pad,), jnp.int32)])
    vals = jnp.concatenate([adj_val.reshape(-1),
                            jnp.zeros((pad,), jnp.float32)])
    cv = jnp.stack([cols.reshape(-1, CH),
                    lax.bitcast_convert_type(vals, jnp.int32).reshape(-1, CH)],
                   axis=1).reshape(-1)
    zeros = jnp.zeros((TPR, SW), jnp.float32)
    emb = ent_emb.reshape(SH * N, SW)
    L = basis_weights.shape[0]
    out = None
    for l in range(L):
        acc = _spmm(rows, cv, emb, zeros)
        res = _transform(acc, basis_weights[l], basis_coeff[l, l],
                         last=(l == L - 1))
        if l == L - 1:
            out = res
        else:
            emb = res.reshape(SH * N, SW)
    return out


# speedup vs baseline: 4.5514x; 1.0646x over previous
import functools

import jax
import jax.numpy as jnp
from jax import lax
from jax.experimental import pallas as pl
from jax.experimental.pallas import tpu as pltpu
from jax.experimental.pallas import tpu_sc as plsc

N = 50000
D = 128
NB = 2

SH = 4
SW = D // SH
TPR = N // 16

NE = 7 * 80000
CH = 128
NCHT = 280
EPT = NCHT * CH
NE_PAD = 16 * EPT
NE_LEN = NE_PAD + 4 * CH
NBUF = 4

ROW_BLK = 400


def _spmm_body(rows_hbm, cv_hbm, emb_hbm, zeros_hbm, acc_hbm, acc_sp,
               *bufs):
    c = lax.axis_index("c")
    s = lax.axis_index("s")
    rows_v = bufs[0:4]
    cv_v = bufs[4:8]
    cidx_v = bufs[8:12]
    srow_v = bufs[12:16]
    gath_v = bufs[16:20]
    msem = bufs[20:24]
    gsem = bufs[24:28]
    ssem = bufs[28:32]
    qbase = s * NCHT
    acc2d = acc_hbm
    zeros2d = zeros_hbm

    def meta_start(b, k):
        q = qbase + k
        pltpu.async_copy(rows_hbm.at[pl.ds(q * CH, CH)], rows_v[b], msem[b])
        pltpu.async_copy(cv_hbm.at[pl.ds(q * 2 * CH, 2 * CH)], cv_v[b], msem[b])

    def meta_wait(b, k):
        q = qbase + k
        pltpu.make_async_copy(
            rows_hbm.at[pl.ds(q * CH, CH)], rows_v[b], msem[b]).wait()
        pltpu.make_async_copy(
            cv_hbm.at[pl.ds(q * 2 * CH, 2 * CH)], cv_v[b], msem[b]).wait()

    def gather_start(b, blkn):
        for g in range(CH // 16):
            sl = pl.ds(g * 16, 16)
            cidx_v[b][sl] = cv_v[b][sl] + blkn
        pltpu.async_copy(emb_hbm.at[cidx_v[b]], gath_v[b], gsem[b])

    def gather_wait(b):
        pltpu.make_async_copy(emb_hbm.at[cidx_v[b]], gath_v[b], gsem[b]).wait()

    def scale_copy(b):
        for g in range(CH // 16):
            sl = pl.ds(g * 16, 16)
            v16 = lax.bitcast_convert_type(cv_v[b][pl.ds(CH + g * 16, 16)],
                                           jnp.float32)
            srow_v[b][sl] = rows_v[b][sl]
            for i in range(16):
                e = g * 16 + i
                mv = v16[i]
                for jj in range(SW // 16):
                    sl2 = pl.ds(jj * 16, 16)
                    gath_v[b][e, sl2] = gath_v[b][e, sl2] * mv

    def scatter_start(b):
        pltpu.async_copy(gath_v[b], acc_sp.at[srow_v[b]], ssem[b], add=True)

    def scatter_wait(b):
        pltpu.make_async_copy(gath_v[b], acc_sp.at[srow_v[b]], ssem[b]).wait()

    def chunk(kk, b, blkn, first=False):
        b2 = (b + 2) % NBUF
        gather_wait(b)
        scale_copy(b)
        scatter_start(b)
        if not first:
            scatter_wait(b2)
        meta_wait(b2, kk + 2)
        gather_start(b2, blkn)
        meta_start(b, kk + 4)

    for p in range(2):
        blk = c * 2 + p
        blkn = blk * N
        pltpu.sync_copy(zeros2d, acc_sp.at[pl.ds(s * TPR, TPR)])
        plsc.subcore_barrier()

        for b in range(NBUF):
            meta_start(b, b)
        for b in range(2):
            meta_wait(b, b)
            gather_start(b, blkn)
        chunk(0, 0, blkn, first=True)
        chunk(1, 1, blkn, first=True)

        def quad_body(i, _):
            kk = 4 * i + 2
            chunk(kk, 2, blkn)
            chunk(kk + 1, 3, blkn)
            chunk(kk + 2, 0, blkn)
            chunk(kk + 3, 1, blkn)
            return 0

        lax.fori_loop(0, (NCHT - 4) // 4, quad_body, 0)

        chunk(NCHT - 2, 2, blkn)
        chunk(NCHT - 1, 3, blkn)
        gather_wait(0)
        gather_wait(1)
        meta_wait(2, NCHT + 2)
        meta_wait(3, NCHT + 3)
        scatter_wait(2)
        scatter_wait(3)

        plsc.subcore_barrier()
        pltpu.sync_copy(acc_sp.at[pl.ds(s * TPR, TPR)],
                        acc2d.at[pl.ds(s * TPR, TPR), pl.ds(blk * SW, SW)])


_spmm = functools.partial(
    pl.kernel,
    out_type=jax.ShapeDtypeStruct((N, D), jnp.float32),
    mesh=plsc.VectorSubcoreMesh(core_axis_name="c", subcore_axis_name="s"),
    compiler_params=pltpu.CompilerParams(use_tc_tiling_on_sc=False),
    scratch_types=(
        [pltpu.VMEM_SHARED((N, SW), jnp.float32)]
        + [pltpu.VMEM((CH,), jnp.int32)] * NBUF
        + [pltpu.VMEM((2 * CH,), jnp.int32)] * NBUF
        + [pltpu.VMEM((CH,), jnp.int32)] * NBUF
        + [pltpu.VMEM((CH,), jnp.int32)] * NBUF
        + [pltpu.VMEM((CH, SW), jnp.float32)] * NBUF
        + [pltpu.SemaphoreType.DMA] * (3 * NBUF)
    ),
)(_spmm_body)


def _transform_body(cvec_ref, w_ref, acc_ref, out_ref, *, last):
    w = cvec_ref[0, 0] * w_ref[0] + cvec_ref[0, 1] * w_ref[1]
    y = jnp.dot(acc_ref[...], w.T, preferred_element_type=jnp.float32)
    y = jnp.maximum(y, 0.0)
    if last:
        nrm = jnp.sqrt(jnp.sum(y * y, axis=1, keepdims=True))
        y = y / jnp.maximum(nrm, 1e-12)
    out_ref[...] = y


def _transform(acc, weights_l, coeff_l, last):
    cvec = coeff_l.reshape(1, NB)
    grid = N // ROW_BLK
    return pl.pallas_call(
        functools.partial(_transform_body, last=last),
        grid=(grid,),
        in_specs=[
            pl.BlockSpec((1, NB), lambda i: (0, 0)),
            pl.BlockSpec((NB, D, D), lambda i: (0, 0, 0)),
            pl.BlockSpec((ROW_BLK, D), lambda i: (i, 0)),
        ],
        out_specs=pl.BlockSpec((ROW_BLK, D), lambda i: (i, 0)),
        out_shape=jax.ShapeDtypeStruct((N, D), jnp.float32),
    )(cvec, weights_l, acc)


def kernel(adj_row, adj_col, adj_val, ent_emb, basis_weights, basis_coeff):
    pad = NE_LEN - NE
    rows = jnp.concatenate([adj_row.reshape(-1).astype(jnp.int32),
                            jnp.zeros((pad,), jnp.int32)])
    cols = jnp.concatenate([adj_col.reshape(-1).astype(jnp.int32),
                            jnp.zeros((pad,), jnp.int32)])
    vals = jnp.concatenate([adj_val.reshape(-1),
                            jnp.zeros((pad,), jnp.float32)])
    cv = jnp.stack([cols.reshape(-1, CH),
                    lax.bitcast_convert_type(vals, jnp.int32).reshape(-1, CH)],
                   axis=1).reshape(-1)
    zeros = jnp.zeros((TPR, SW), jnp.float32)
    emb = ent_emb.reshape(N, SH, SW).transpose(1, 0, 2).reshape(SH * N, SW)
    L = basis_weights.shape[0]
    out = None
    for l in range(L):
        acc = _spmm(rows, cv, emb, zeros)
        res = _transform(acc, basis_weights[l], basis_coeff[l, l],
                         last=(l == L - 1))
        if l == L - 1:
            out = res
        else:
            emb = res.reshape(N, SH, SW).transpose(1, 0, 2).reshape(SH * N, SW)
    return out


# speedup vs baseline: 5.1284x; 1.1268x over previous
import functools

import jax
import jax.numpy as jnp
from jax import lax
from jax.experimental import pallas as pl
from jax.experimental.pallas import tpu as pltpu
from jax.experimental.pallas import tpu_sc as plsc

N = 50000
D = 128
NB = 2

SH = 4
SW = D // SH
TPR = N // 16

NE = 7 * 80000
CH = 128
NCHT = 280
EPT = NCHT * CH
NE_PAD = 16 * EPT
NE_LEN = NE_PAD + 5 * CH
NBUF = 5

ROW_BLK = 2000


def _spmm_body(rows_hbm, cv_hbm, emb_hbm, zeros_hbm, acc_hbm, acc_sp,
               *bufs):
    c = lax.axis_index("c")
    s = lax.axis_index("s")
    rows_v = bufs[0 * NBUF:1 * NBUF]
    cv_v = bufs[1 * NBUF:2 * NBUF]
    cidx_v = bufs[2 * NBUF:3 * NBUF]
    srow_v = bufs[3 * NBUF:4 * NBUF]
    gath_v = bufs[4 * NBUF:5 * NBUF]
    msem = bufs[5 * NBUF:6 * NBUF]
    gsem = bufs[6 * NBUF:7 * NBUF]
    ssem = bufs[7 * NBUF:8 * NBUF]
    qbase = s * NCHT
    acc2d = acc_hbm
    zeros2d = zeros_hbm

    def meta_start(b, k):
        q = qbase + k
        pltpu.async_copy(rows_hbm.at[pl.ds(q * CH, CH)], rows_v[b], msem[b])
        pltpu.async_copy(cv_hbm.at[pl.ds(q * 2 * CH, 2 * CH)], cv_v[b], msem[b])

    def meta_wait(b, k):
        q = qbase + k
        pltpu.make_async_copy(
            rows_hbm.at[pl.ds(q * CH, CH)], rows_v[b], msem[b]).wait()
        pltpu.make_async_copy(
            cv_hbm.at[pl.ds(q * 2 * CH, 2 * CH)], cv_v[b], msem[b]).wait()

    def gather_start(b, blkn):
        for g in range(CH // 16):
            sl = pl.ds(g * 16, 16)
            cidx_v[b][sl] = cv_v[b][sl] + blkn
        pltpu.async_copy(emb_hbm.at[cidx_v[b]], gath_v[b], gsem[b])

    def gather_wait(b):
        pltpu.make_async_copy(emb_hbm.at[cidx_v[b]], gath_v[b], gsem[b]).wait()

    def scale_copy(b):
        for g in range(CH // 16):
            sl = pl.ds(g * 16, 16)
            v16 = lax.bitcast_convert_type(cv_v[b][pl.ds(CH + g * 16, 16)],
                                           jnp.float32)
            srow_v[b][sl] = rows_v[b][sl]
            for i in range(16):
                e = g * 16 + i
                mv = v16[i]
                for jj in range(SW // 16):
                    sl2 = pl.ds(jj * 16, 16)
                    gath_v[b][e, sl2] = gath_v[b][e, sl2] * mv

    def scatter_start(b):
        pltpu.async_copy(gath_v[b], acc_sp.at[srow_v[b]], ssem[b], add=True)

    def scatter_wait(b):
        pltpu.make_async_copy(gath_v[b], acc_sp.at[srow_v[b]], ssem[b]).wait()

    def chunk(kk, b, blkn, first=False):
        b3 = (b + 3) % NBUF
        gather_wait(b)
        scale_copy(b)
        scatter_start(b)
        if not first:
            scatter_wait(b3)
        meta_wait(b3, kk + 3)
        gather_start(b3, blkn)
        meta_start(b, kk + 5)

    for p in range(2):
        blk = c * 2 + p
        blkn = blk * N
        pltpu.sync_copy(zeros2d, acc_sp.at[pl.ds(s * TPR, TPR)])
        plsc.subcore_barrier()

        for b in range(NBUF):
            meta_start(b, b)
        for b in range(3):
            meta_wait(b, b)
            gather_start(b, blkn)
        chunk(0, 0, blkn, first=True)
        chunk(1, 1, blkn, first=True)
        chunk(2, 2, blkn, first=True)

        def quint_body(i, _):
            kk = 5 * i + 3
            chunk(kk, 3, blkn)
            chunk(kk + 1, 4, blkn)
            chunk(kk + 2, 0, blkn)
            chunk(kk + 3, 1, blkn)
            chunk(kk + 4, 2, blkn)
            return 0

        lax.fori_loop(0, (NCHT - 5) // 5, quint_body, 0)

        chunk(NCHT - 2, 3, blkn)
        chunk(NCHT - 1, 4, blkn)
        gather_wait(0)
        gather_wait(1)
        gather_wait(2)
        meta_wait(3, NCHT + 3)
        meta_wait(4, NCHT + 4)
        scatter_wait(3)
        scatter_wait(4)

        plsc.subcore_barrier()
        pltpu.sync_copy(acc_sp.at[pl.ds(s * TPR, TPR)],
                        acc2d.at[pl.ds(s * TPR, TPR), pl.ds(blk * SW, SW)])


_spmm = functools.partial(
    pl.kernel,
    out_type=jax.ShapeDtypeStruct((N, D), jnp.float32),
    mesh=plsc.VectorSubcoreMesh(core_axis_name="c", subcore_axis_name="s"),
    compiler_params=pltpu.CompilerParams(use_tc_tiling_on_sc=False),
    scratch_types=(
        [pltpu.VMEM_SHARED((N, SW), jnp.float32)]
        + [pltpu.VMEM((CH,), jnp.int32)] * NBUF
        + [pltpu.VMEM((2 * CH,), jnp.int32)] * NBUF
        + [pltpu.VMEM((CH,), jnp.int32)] * NBUF
        + [pltpu.VMEM((CH,), jnp.int32)] * NBUF
        + [pltpu.VMEM((CH, SW), jnp.float32)] * NBUF
        + [pltpu.SemaphoreType.DMA] * (3 * NBUF)
    ),
)(_spmm_body)


def _transform_body(cvec_ref, w_ref, acc_ref, out_ref, *, last):
    w = cvec_ref[0, 0] * w_ref[0] + cvec_ref[0, 1] * w_ref[1]
    y = jnp.dot(acc_ref[...], w.T, preferred_element_type=jnp.float32)
    y = jnp.maximum(y, 0.0)
    if last:
        nrm = jnp.sqrt(jnp.sum(y * y, axis=1, keepdims=True))
        y = y / jnp.maximum(nrm, 1e-12)
    out_ref[...] = y


def _transform(acc, weights_l, coeff_l, last):
    cvec = coeff_l.reshape(1, NB)
    grid = N // ROW_BLK
    return pl.pallas_call(
        functools.partial(_transform_body, last=last),
        grid=(grid,),
        in_specs=[
            pl.BlockSpec((1, NB), lambda i: (0, 0)),
            pl.BlockSpec((NB, D, D), lambda i: (0, 0, 0)),
            pl.BlockSpec((ROW_BLK, D), lambda i: (i, 0)),
        ],
        out_specs=pl.BlockSpec((ROW_BLK, D), lambda i: (i, 0)),
        out_shape=jax.ShapeDtypeStruct((N, D), jnp.float32),
    )(cvec, weights_l, acc)


def kernel(adj_row, adj_col, adj_val, ent_emb, basis_weights, basis_coeff):
    pad = NE_LEN - NE
    rows = jnp.concatenate([adj_row.reshape(-1).astype(jnp.int32),
                            jnp.zeros((pad,), jnp.int32)])
    cols = jnp.concatenate([adj_col.reshape(-1).astype(jnp.int32),
                            jnp.zeros((pad,), jnp.int32)])
    vals = jnp.concatenate([adj_val.reshape(-1),
                            jnp.zeros((pad,), jnp.float32)])
    cv = jnp.stack([cols.reshape(-1, CH),
                    lax.bitcast_convert_type(vals, jnp.int32).reshape(-1, CH)],
                   axis=1).reshape(-1)
    zeros = jnp.zeros((TPR, SW), jnp.float32)
    emb = ent_emb.reshape(N, SH, SW).transpose(1, 0, 2).reshape(SH * N, SW)
    L = basis_weights.shape[0]
    out = None
    for l in range(L):
        acc = _spmm(rows, cv, emb, zeros)
        res = _transform(acc, basis_weights[l], basis_coeff[l, l],
                         last=(l == L - 1))
        if l == L - 1:
            out = res
        else:
            emb = res.reshape(N, SH, SW).transpose(1, 0, 2).reshape(SH * N, SW)
    return out


# speedup vs baseline: 5.3250x; 1.0383x over previous
import functools

import jax
import jax.numpy as jnp
from jax import lax
from jax.experimental import pallas as pl
from jax.experimental.pallas import tpu as pltpu
from jax.experimental.pallas import tpu_sc as plsc

N = 50000
D = 128
NB = 2

SH = 4
SW = D // SH
TPR = N // 16

NE = 7 * 80000
CH = 128
NCHT = 280
EPT = NCHT * CH
NE_PAD = 16 * EPT
NE_LEN = NE_PAD + 7 * CH
NBUF = 5

ROW_BLK = 2000


def _spmm_body(rows_hbm, cv_hbm, emb_hbm, zeros_hbm, acc_hbm, acc_sp,
               *bufs):
    c = lax.axis_index("c")
    s = lax.axis_index("s")
    rows_v = bufs[0 * NBUF:1 * NBUF]
    cv_v = bufs[1 * NBUF:2 * NBUF]
    cidx_v = bufs[2 * NBUF:3 * NBUF]
    srow_v = bufs[3 * NBUF:4 * NBUF]
    gath_v = bufs[4 * NBUF:5 * NBUF]
    msem = bufs[5 * NBUF:6 * NBUF]
    gsem = bufs[6 * NBUF:7 * NBUF]
    ssem = bufs[7 * NBUF:8 * NBUF]
    scal_v = bufs[8 * NBUF:8 * NBUF + 2]
    qbase = s * NCHT
    acc2d = acc_hbm
    zeros2d = zeros_hbm

    def meta_start(b, k):
        q = qbase + k
        pltpu.async_copy(rows_hbm.at[pl.ds(q * CH, CH)], rows_v[b], msem[b])
        pltpu.async_copy(cv_hbm.at[pl.ds(q * 2 * CH, 2 * CH)], cv_v[b], msem[b])

    def meta_wait(b, k):
        q = qbase + k
        pltpu.make_async_copy(
            rows_hbm.at[pl.ds(q * CH, CH)], rows_v[b], msem[b]).wait()
        pltpu.make_async_copy(
            cv_hbm.at[pl.ds(q * 2 * CH, 2 * CH)], cv_v[b], msem[b]).wait()

    def gather_start(b, blkn):
        for g in range(CH // 16):
            sl = pl.ds(g * 16, 16)
            cidx_v[b][sl] = cv_v[b][sl] + blkn
        pltpu.async_copy(emb_hbm.at[cidx_v[b]], gath_v[b], gsem[b])

    def gather_wait(b):
        pltpu.make_async_copy(emb_hbm.at[cidx_v[b]], gath_v[b], gsem[b]).wait()

    def scale_copy(b, m):
        def sgroup(g, _):
            sl = pl.ds(g * 16, 16)
            v16 = lax.bitcast_convert_type(cv_v[b][pl.ds(CH + g * 16, 16)],
                                           jnp.float32)
            srow_v[b][sl] = rows_v[b][sl]
            for i in range(16):
                e = g * 16 + i
                mv = v16[i]
                w = gath_v[b][e, :]
                lo = lax.bitcast_convert_type(w << 16, jnp.float32)
                hi = lax.bitcast_convert_type(
                    w & jnp.int32(-65536), jnp.float32)
                scal_v[m][e, pl.ds(0, 16)] = lo * mv
                scal_v[m][e, pl.ds(16, 16)] = hi * mv
            return 0

        lax.fori_loop(0, CH // 16, sgroup, 0)

    def scatter_start(b, m):
        pltpu.async_copy(scal_v[m], acc_sp.at[srow_v[b]], ssem[b], add=True)

    def scatter_wait(b, m):
        pltpu.make_async_copy(scal_v[m], acc_sp.at[srow_v[b]],
                              ssem[b]).wait()

    LOOK = 3

    def chunk(kk, b, m, blkn, first=False):
        bl = (b + LOOK) % NBUF
        gather_wait(b)
        if not first:
            scatter_wait(bl, m)
        scale_copy(b, m)
        scatter_start(b, m)
        meta_wait(bl, kk + LOOK)
        gather_start(bl, blkn)
        meta_start(b, kk + NBUF)

    def one_pass(p, _):
        blk = c * 2 + p
        blkn = blk * N
        pltpu.sync_copy(zeros2d, acc_sp.at[pl.ds(s * TPR, TPR)])
        plsc.subcore_barrier()

        for b in range(NBUF):
            meta_start(b, b)
        for b in range(LOOK):
            meta_wait(b, b)
            gather_start(b, blkn)
        chunk(0, 0, 0, blkn, first=True)
        chunk(1, 1, 1, blkn, first=True)
        chunk(2, 2, 0, blkn)

        def dec_body(i, _):
            kk = 10 * i + 3
            for j in range(10):
                chunk(kk + j, (3 + j) % NBUF, (3 + j) % 2, blkn)
            return 0

        lax.fori_loop(0, 27, dec_body, 0)

        for j in range(7):
            kk = 273 + j
            chunk(kk, kk % NBUF, kk % 2, blkn)
        for b in range(LOOK):
            gather_wait(b)
        meta_wait(3, NCHT + 3)
        meta_wait(4, NCHT + 4)
        scatter_wait(3, 0)
        scatter_wait(4, 1)

        plsc.subcore_barrier()
        pltpu.sync_copy(acc_sp.at[pl.ds(s * TPR, TPR)],
                        acc2d.at[pl.ds(s * TPR, TPR), pl.ds(blk * SW, SW)])
        return 0

    lax.fori_loop(0, 2, one_pass, 0)


_spmm = functools.partial(
    pl.kernel,
    out_type=jax.ShapeDtypeStruct((N, D), jnp.float32),
    mesh=plsc.VectorSubcoreMesh(core_axis_name="c", subcore_axis_name="s"),
    compiler_params=pltpu.CompilerParams(use_tc_tiling_on_sc=False),
    scratch_types=(
        [pltpu.VMEM_SHARED((N, SW), jnp.float32)]
        + [pltpu.VMEM((CH,), jnp.int32)] * NBUF
        + [pltpu.VMEM((2 * CH,), jnp.int32)] * NBUF
        + [pltpu.VMEM((CH,), jnp.int32)] * NBUF
        + [pltpu.VMEM((CH,), jnp.int32)] * NBUF
        + [pltpu.VMEM((CH, SW // 2), jnp.int32)] * NBUF
        + [pltpu.SemaphoreType.DMA] * (3 * NBUF)
        + [pltpu.VMEM((CH, SW), jnp.float32)] * 2
    ),
)(_spmm_body)


def _transform_body(cvec_ref, w_ref, acc_ref, out_ref, *, last):
    w = cvec_ref[0, 0] * w_ref[0] + cvec_ref[0, 1] * w_ref[1]
    y = jnp.dot(acc_ref[...], w.T, preferred_element_type=jnp.float32)
    y = jnp.maximum(y, 0.0)
    if last:
        nrm = jnp.sqrt(jnp.sum(y * y, axis=1, keepdims=True))
        y = y / jnp.maximum(nrm, 1e-12)
    out_ref[...] = y


def _transform(acc, weights_l, coeff_l, last):
    cvec = coeff_l.reshape(1, NB)
    grid = N // ROW_BLK
    return pl.pallas_call(
        functools.partial(_transform_body, last=last),
        grid=(grid,),
        in_specs=[
            pl.BlockSpec((1, NB), lambda i: (0, 0)),
            pl.BlockSpec((NB, D, D), lambda i: (0, 0, 0)),
            pl.BlockSpec((ROW_BLK, D), lambda i: (i, 0)),
        ],
        out_specs=pl.BlockSpec((ROW_BLK, D), lambda i: (i, 0)),
        out_shape=jax.ShapeDtypeStruct((N, D), jnp.float32),
    )(cvec, weights_l, acc)


def _to_table(x):
    t = x.astype(jnp.bfloat16).reshape(N, SH, 2, 16)
    t = t.transpose(1, 0, 3, 2).reshape(SH * N, SW // 2, 2)
    return lax.bitcast_convert_type(t, jnp.int32)


def kernel(adj_row, adj_col, adj_val, ent_emb, basis_weights, basis_coeff):
    pad = NE_LEN - NE
    rows = jnp.concatenate([adj_row.reshape(-1).astype(jnp.int32),
                            jnp.zeros((pad,), jnp.int32)])
    cols = jnp.concatenate([adj_col.reshape(-1).astype(jnp.int32),
                            jnp.zeros((pad,), jnp.int32)])
    vals = jnp.concatenate([adj_val.reshape(-1),
                            jnp.zeros((pad,), jnp.float32)])
    cv = jnp.stack([cols.reshape(-1, CH),
                    lax.bitcast_convert_type(vals, jnp.int32).reshape(-1, CH)],
                   axis=1).reshape(-1)
    zeros = jnp.zeros((TPR, SW), jnp.float32)
    emb = _to_table(ent_emb)
    L = basis_weights.shape[0]
    out = None
    for l in range(L):
        acc = _spmm(rows, cv, emb, zeros)
        res = _transform(acc, basis_weights[l], basis_coeff[l, l],
                         last=(l == L - 1))
        if l == L - 1:
            out = res
        else:
            emb = _to_table(res)
    return out
